# Initial kernel scaffold; baseline (speedup 1.0000x reference)
import jax
import jax.numpy as jnp
from jax.experimental import pallas as pl


def kernel(inputs, embedding):
    raise NotImplementedError("write your pallas kernel here")


# speedup vs baseline: 1.6831x; 1.6831x over previous
import functools

import jax
import jax.numpy as jnp
from jax import lax
from jax.experimental import pallas as pl
from jax.experimental.pallas import tpu as pltpu
from jax.experimental.pallas import tpu_sc as plsc

_D = 64
_NC = 2
_NS = 16
_NW = _NC * _NS
_CHUNK = 128


@functools.lru_cache(maxsize=None)
def _make_gather(n_total: int):
    per_w = n_total // _NW
    n_chunk = per_w // _CHUNK
    mesh = plsc.VectorSubcoreMesh(core_axis_name="c", subcore_axis_name="s")

    @functools.partial(
        pl.kernel,
        mesh=mesh,
        out_type=jax.ShapeDtypeStruct((n_total, _D), jnp.float32),
        scratch_types=[
            pltpu.VMEM((n_chunk, _CHUNK), jnp.int32),
            pltpu.VMEM((2, _CHUNK, _D), jnp.float32),
            pltpu.SemaphoreType.DMA,
        ],
        compiler_params=pltpu.CompilerParams(use_tc_tiling_on_sc=False),
    )
    def k(idx_hbm, table_hbm, out_hbm, idx_v, rows_v, sem):
        wid = lax.axis_index("s") * _NC + lax.axis_index("c")
        base = wid * per_w
        pltpu.sync_copy(idx_hbm.at[wid], idx_v)

        def body(j, _):
            pltpu.async_copy(table_hbm.at[idx_v.at[j]], rows_v.at[0], sem).wait()
            pltpu.sync_copy(rows_v.at[0], out_hbm.at[pl.ds(base + j * _CHUNK, _CHUNK)])
            return 0

        lax.fori_loop(0, n_chunk, body, 0)

    return k


def kernel(inputs, embedding):
    b, h = inputs.shape
    n = b * h
    idx = inputs.reshape(_NW, n // _NW // _CHUNK, _CHUNK).astype(jnp.int32)
    out = _make_gather(n)(idx, embedding)
    return out.reshape(b, h, _D)


# speedup vs baseline: 1.8785x; 1.1161x over previous
import functools

import jax
import jax.numpy as jnp
from jax import lax
from jax.experimental import pallas as pl
from jax.experimental.pallas import tpu as pltpu
from jax.experimental.pallas import tpu_sc as plsc

_D = 64
_NC = 2
_NS = 16
_NW = _NC * _NS
_CHUNK = 128
_NBUF = 4
_LEAD = 2


@functools.lru_cache(maxsize=None)
def _make_gather(n_total: int):
    per_w = n_total // _NW
    n_chunk = per_w // _CHUNK
    assert n_chunk % _NBUF == 0 and n_chunk >= 2 * _NBUF
    mesh = plsc.VectorSubcoreMesh(core_axis_name="c", subcore_axis_name="s")

    @functools.partial(
        pl.kernel,
        mesh=mesh,
        out_type=jax.ShapeDtypeStruct((n_total, _D), jnp.float32),
        scratch_types=[
            pltpu.VMEM((n_chunk, _CHUNK), jnp.int32),
            pltpu.VMEM((_NBUF, _CHUNK, _D), jnp.float32),
            pltpu.SemaphoreType.DMA((_NBUF,)),
            pltpu.SemaphoreType.DMA((_NBUF,)),
        ],
        compiler_params=pltpu.CompilerParams(use_tc_tiling_on_sc=False),
    )
    def k(idx_hbm, table_hbm, out_hbm, idx_v, rows_v, gsem, ssem):
        wid = lax.axis_index("s") * _NC + lax.axis_index("c")
        base = wid * per_w
        pltpu.sync_copy(idx_hbm.at[wid], idx_v)

        def fire_gather(j, b):
            pltpu.async_copy(table_hbm.at[idx_v.at[j]], rows_v.at[b], gsem.at[b])

        def wait_gather(b):
            pltpu.make_async_copy(
                table_hbm.at[pl.ds(0, _CHUNK)], rows_v.at[b], gsem.at[b]
            ).wait()

        def fire_store(j, b):
            pltpu.async_copy(
                rows_v.at[b], out_hbm.at[pl.ds(base + j * _CHUNK, _CHUNK)], ssem.at[b]
            )

        def wait_store(b):
            pltpu.make_async_copy(
                rows_v.at[b], out_hbm.at[pl.ds(base, _CHUNK)], ssem.at[b]
            ).wait()

        for j in range(_LEAD):
            fire_gather(j, j % _NBUF)
        for j in range(_NBUF):
            b = j % _NBUF
            b2 = (j + _LEAD) % _NBUF
            if j + _LEAD >= _NBUF:
                wait_store(b2)
            fire_gather(j + _LEAD, b2)
            wait_gather(b)
            fire_store(j, b)

        def body(outer, _):
            for b in range(_NBUF):
                j = outer * _NBUF + b
                b2 = (b + _LEAD) % _NBUF
                wait_store(b2)
                fire_gather(j + _LEAD, b2)
                wait_gather(b)
                fire_store(j, b)
            return 0

        lax.fori_loop(1, n_chunk // _NBUF - 1, body, 0, unroll=False)

        for j in range(n_chunk - _NBUF, n_chunk):
            b = j % _NBUF
            b2 = (j + _LEAD) % _NBUF
            if j + _LEAD < n_chunk:
                wait_store(b2)
                fire_gather(j + _LEAD, b2)
            wait_gather(b)
            fire_store(j, b)
        for b in range(_NBUF):
            wait_store(b)

    return k


def kernel(inputs, embedding):
    b, h = inputs.shape
    n = b * h
    idx = inputs.reshape(_NW, n // _NW // _CHUNK, _CHUNK).astype(jnp.int32)
    out = _make_gather(n)(idx, embedding)
    return out.reshape(b, h, _D)
